# Initial kernel scaffold; baseline (speedup 1.0000x reference)
import jax
import jax.numpy as jnp
from jax.experimental import pallas as pl


def kernel(hidden_states, gate_w, gate_proj_w, up_proj_w, down_proj_w):
    raise NotImplementedError("write your pallas kernel here")


# speedup vs baseline: 1.0738x; 1.0738x over previous
import functools

import jax
import jax.numpy as jnp
from jax import lax
from jax.experimental import pallas as pl
from jax.experimental.pallas import tpu as pltpu
from jax.experimental.pallas import tpu_sc as plsc

NE = 8
TK = 2
BLK = 128
S = 2048
H = 768
I = 1536
NSLOT = S * TK
PAD = NSLOT + NE * BLK
NB = PAD // BLK


def _router_body(x_ref, gw_ref, logits_ref, ids_ref, ws_ref):
    x = x_ref[...]
    gw = gw_ref[...]
    logits = lax.dot_general(x, gw, (((1,), (1,)), ((), ())),
                             preferred_element_type=jnp.float32)
    logits_ref[...] = logits
    m = jnp.max(logits, axis=-1, keepdims=True)
    ex = jnp.exp(logits - m)
    p = ex / jnp.sum(ex, axis=-1, keepdims=True)
    idx = lax.broadcasted_iota(jnp.int32, p.shape, 1)
    m1 = jnp.max(p, axis=-1, keepdims=True)
    i1 = jnp.min(jnp.where(p == m1, idx, NE), axis=-1, keepdims=True)
    p2 = jnp.where(idx == i1, -1.0, p)
    m2 = jnp.max(p2, axis=-1, keepdims=True)
    i2 = jnp.min(jnp.where(p2 == m2, idx, NE), axis=-1, keepdims=True)
    denom = m1 + m2
    w1 = m1 / denom
    w2 = m2 / denom
    ids_ref[...] = jnp.where(idx == 0, i1, jnp.where(idx == 1, i2, 0))
    ws_ref[...] = jnp.where(idx == 0, w1, jnp.where(idx == 1, w2, 0.0))


def _run_router(x, gate_w):
    return pl.pallas_call(
        _router_body,
        out_shape=(
            jax.ShapeDtypeStruct((S, NE), jnp.float32),
            jax.ShapeDtypeStruct((S, NE), jnp.int32),
            jax.ShapeDtypeStruct((S, NE), jnp.float32),
        ),
    )(x, gate_w)


def _gmm_body(be_ref, xg_ref, gw_ref, uw_ref, dw_ref, wcol_ref, out_ref):
    del be_ref
    x = xg_ref[...]
    g = lax.dot_general(x, gw_ref[0], (((1,), (1,)), ((), ())),
                        preferred_element_type=jnp.float32)
    u = lax.dot_general(x, uw_ref[0], (((1,), (1,)), ((), ())),
                        preferred_element_type=jnp.float32)
    h = g * jax.nn.sigmoid(g) * u
    y = lax.dot_general(h, dw_ref[0], (((1,), (1,)), ((), ())),
                        preferred_element_type=jnp.float32)
    out_ref[...] = y * wcol_ref[...][:, :1]


def _run_gmm(block_expert, xg, gate_proj_w, up_proj_w, down_proj_w, wcol):
    grid_spec = pltpu.PrefetchScalarGridSpec(
        num_scalar_prefetch=1,
        grid=(NB,),
        in_specs=[
            pl.BlockSpec((BLK, H), lambda i, be: (i, 0)),
            pl.BlockSpec((1, I, H), lambda i, be: (be[i], 0, 0)),
            pl.BlockSpec((1, I, H), lambda i, be: (be[i], 0, 0)),
            pl.BlockSpec((1, H, I), lambda i, be: (be[i], 0, 0)),
            pl.BlockSpec((BLK, 128), lambda i, be: (i, 0)),
        ],
        out_specs=pl.BlockSpec((BLK, H), lambda i, be: (i, 0)),
    )
    return pl.pallas_call(
        _gmm_body,
        grid_spec=grid_spec,
        out_shape=jax.ShapeDtypeStruct((PAD, H), jnp.float32),
    )(block_expert, xg, gate_proj_w, up_proj_w, down_proj_w, wcol)


def _sc_gather(table, idx):
    mesh = plsc.VectorSubcoreMesh(core_axis_name="c", subcore_axis_name="s")
    nw = mesh.num_cores * mesh.num_subcores
    n = idx.shape[0]
    b_per_w = n // nw

    @functools.partial(
        pl.kernel,
        out_type=jax.ShapeDtypeStruct((n, H), jnp.float32),
        mesh=mesh,
        scratch_types=[
            pltpu.VMEM((b_per_w,), jnp.int32),
            pltpu.VMEM((b_per_w, H), jnp.float32),
            pltpu.SemaphoreType.DMA,
        ],
    )
    def k(table_hbm, idx_hbm, out_hbm, idx_v, rows_v, sem):
        wid = lax.axis_index("s") * mesh.num_cores + lax.axis_index("c")
        base = wid * b_per_w
        pltpu.sync_copy(idx_hbm.at[pl.ds(base, b_per_w)], idx_v)
        pltpu.async_copy(table_hbm.at[idx_v], rows_v, sem).wait()
        pltpu.sync_copy(rows_v, out_hbm.at[pl.ds(base, b_per_w)])

    return k(table, idx)


def _sc_combine(osort, pos0, pos1):
    mesh = plsc.VectorSubcoreMesh(core_axis_name="c", subcore_axis_name="s")
    nw = mesh.num_cores * mesh.num_subcores
    t_per_w = S // nw
    csteps = H // 16

    @functools.partial(
        pl.kernel,
        out_type=jax.ShapeDtypeStruct((S, H), jnp.float32),
        mesh=mesh,
        scratch_types=[
            pltpu.VMEM((t_per_w,), jnp.int32),
            pltpu.VMEM((t_per_w,), jnp.int32),
            pltpu.VMEM((t_per_w, H), jnp.float32),
            pltpu.VMEM((t_per_w, H), jnp.float32),
            pltpu.SemaphoreType.DMA,
        ],
    )
    def k(osort_hbm, p0_hbm, p1_hbm, out_hbm, i0_v, i1_v, a_v, b_v, sem):
        wid = lax.axis_index("s") * mesh.num_cores + lax.axis_index("c")
        base = wid * t_per_w
        pltpu.sync_copy(p0_hbm.at[pl.ds(base, t_per_w)], i0_v)
        pltpu.sync_copy(p1_hbm.at[pl.ds(base, t_per_w)], i1_v)
        pltpu.async_copy(osort_hbm.at[i0_v], a_v, sem).wait()
        pltpu.async_copy(osort_hbm.at[i1_v], b_v, sem).wait()

        def row(r, carry):
            def col(c, carry2):
                sl = pl.ds(c * 16, 16)
                a_v[r, sl] = a_v[r, sl] + b_v[r, sl]
                return carry2
            return lax.fori_loop(0, csteps, col, carry)

        lax.fori_loop(0, t_per_w, row, 0)
        pltpu.sync_copy(a_v, out_hbm.at[pl.ds(base, t_per_w)])

    return k(osort, pos0, pos1)


def kernel(hidden_states, gate_w, gate_proj_w, up_proj_w, down_proj_w):
    batch, seq, hdim = hidden_states.shape
    x = hidden_states.reshape(-1, hdim)

    logits, ids8, ws8 = _run_router(x, gate_w)
    e2 = ids8[:, :TK]
    w2 = ws8[:, :TK]

    e_flat = e2.reshape(-1)
    onehot = (e_flat[:, None] == jnp.arange(NE, dtype=jnp.int32)[None, :])
    within = jnp.cumsum(onehot.astype(jnp.int32), axis=0)
    rank = jnp.take_along_axis(within, e_flat[:, None], axis=1)[:, 0] - 1
    counts = within[-1]
    padded = ((counts + BLK - 1) // BLK) * BLK
    pad_end = jnp.cumsum(padded)
    pad_off = pad_end - padded
    pos = pad_off[e_flat] + rank

    slot_token = jnp.arange(NSLOT, dtype=jnp.int32) // TK
    row_ids = jnp.zeros((PAD,), jnp.int32).at[pos].set(slot_token)
    w_pad = jnp.zeros((PAD,), jnp.float32).at[pos].set(w2.reshape(-1))
    block_expert = jnp.searchsorted(
        pad_end, jnp.arange(NB, dtype=jnp.int32) * BLK, side="right")
    block_expert = jnp.minimum(block_expert, NE - 1).astype(jnp.int32)

    xg = _sc_gather(x, row_ids)
    wcol = jnp.broadcast_to(w_pad[:, None], (PAD, 128))
    osort = _run_gmm(block_expert, xg, gate_proj_w, up_proj_w,
                     down_proj_w, wcol)
    pos2 = pos.reshape(S, TK)
    final = _sc_combine(osort, pos2[:, 0], pos2[:, 1])
    return final.reshape(batch, seq, hdim), logits


# speedup vs baseline: 1.0743x; 1.0005x over previous
import functools

import jax
import jax.numpy as jnp
from jax import lax
from jax.experimental import pallas as pl
from jax.experimental.pallas import tpu as pltpu
from jax.experimental.pallas import tpu_sc as plsc

NE = 8
TK = 2
BLK = 128
S = 2048
H = 768
I = 1536
NSLOT = S * TK
PAD = NSLOT + NE * BLK
NB = PAD // BLK


def _router_body(x_ref, gw_ref, logits_ref, ids_ref, ws_ref):
    x = x_ref[...]
    gw = gw_ref[...]
    logits = lax.dot_general(x, gw, (((1,), (1,)), ((), ())),
                             preferred_element_type=jnp.float32)
    logits_ref[...] = logits
    m = jnp.max(logits, axis=-1, keepdims=True)
    ex = jnp.exp(logits - m)
    p = ex / jnp.sum(ex, axis=-1, keepdims=True)
    idx = lax.broadcasted_iota(jnp.int32, p.shape, 1)
    m1 = jnp.max(p, axis=-1, keepdims=True)
    i1 = jnp.min(jnp.where(p == m1, idx, NE), axis=-1, keepdims=True)
    p2 = jnp.where(idx == i1, -1.0, p)
    m2 = jnp.max(p2, axis=-1, keepdims=True)
    i2 = jnp.min(jnp.where(p2 == m2, idx, NE), axis=-1, keepdims=True)
    denom = m1 + m2
    w1 = m1 / denom
    w2 = m2 / denom
    ids_ref[...] = jnp.where(idx == 0, i1, jnp.where(idx == 1, i2, 0))
    ws_ref[...] = jnp.where(idx == 0, w1, jnp.where(idx == 1, w2, 0.0))


def _run_router(x, gate_w):
    return pl.pallas_call(
        _router_body,
        out_shape=(
            jax.ShapeDtypeStruct((S, NE), jnp.float32),
            jax.ShapeDtypeStruct((S, NE), jnp.int32),
            jax.ShapeDtypeStruct((S, NE), jnp.float32),
        ),
    )(x, gate_w)


def _gmm_body(be_ref, xg_ref, gw_ref, uw_ref, dw_ref, wcol_ref, out_ref):
    del be_ref
    x = xg_ref[...]
    g = lax.dot_general(x, gw_ref[0], (((1,), (1,)), ((), ())),
                        preferred_element_type=jnp.float32)
    u = lax.dot_general(x, uw_ref[0], (((1,), (1,)), ((), ())),
                        preferred_element_type=jnp.float32)
    h = g * jax.nn.sigmoid(g) * u
    y = lax.dot_general(h, dw_ref[0], (((1,), (1,)), ((), ())),
                        preferred_element_type=jnp.float32)
    out_ref[...] = y * wcol_ref[...][:, :1]


def _run_gmm(block_expert, xg, gate_proj_w, up_proj_w, down_proj_w, wcol):
    grid_spec = pltpu.PrefetchScalarGridSpec(
        num_scalar_prefetch=1,
        grid=(NB,),
        in_specs=[
            pl.BlockSpec((BLK, H), lambda i, be: (i, 0)),
            pl.BlockSpec((1, I, H), lambda i, be: (be[i], 0, 0)),
            pl.BlockSpec((1, I, H), lambda i, be: (be[i], 0, 0)),
            pl.BlockSpec((1, H, I), lambda i, be: (be[i], 0, 0)),
            pl.BlockSpec((BLK, 128), lambda i, be: (i, 0)),
        ],
        out_specs=pl.BlockSpec((BLK, H), lambda i, be: (i, 0)),
    )
    return pl.pallas_call(
        _gmm_body,
        grid_spec=grid_spec,
        out_shape=jax.ShapeDtypeStruct((PAD, H), jnp.float32),
    )(block_expert, xg, gate_proj_w, up_proj_w, down_proj_w, wcol)


def _sc_gather(table, idx):
    mesh = plsc.VectorSubcoreMesh(core_axis_name="c", subcore_axis_name="s")
    nw = mesh.num_cores * mesh.num_subcores
    n = idx.shape[0]
    b_per_w = n // nw

    @functools.partial(
        pl.kernel,
        out_type=jax.ShapeDtypeStruct((n, H), jnp.float32),
        mesh=mesh,
        name="sc_gather_rows",
        scratch_types=[
            pltpu.VMEM((b_per_w,), jnp.int32),
            pltpu.VMEM((b_per_w, H), jnp.float32),
            pltpu.SemaphoreType.DMA,
        ],
    )
    def k(table_hbm, idx_hbm, out_hbm, idx_v, rows_v, sem):
        wid = lax.axis_index("s") * mesh.num_cores + lax.axis_index("c")
        base = wid * b_per_w
        pltpu.sync_copy(idx_hbm.at[pl.ds(base, b_per_w)], idx_v)
        pltpu.async_copy(table_hbm.at[idx_v], rows_v, sem).wait()
        pltpu.sync_copy(rows_v, out_hbm.at[pl.ds(base, b_per_w)])

    return k(table, idx)


def _sc_combine(osort, pos0, pos1):
    mesh = plsc.VectorSubcoreMesh(core_axis_name="c", subcore_axis_name="s")
    nw = mesh.num_cores * mesh.num_subcores
    t_per_w = S // nw
    csteps = H // 16

    @functools.partial(
        pl.kernel,
        out_type=jax.ShapeDtypeStruct((S, H), jnp.float32),
        mesh=mesh,
        name="sc_combine_rows",
        scratch_types=[
            pltpu.VMEM((t_per_w,), jnp.int32),
            pltpu.VMEM((t_per_w,), jnp.int32),
            pltpu.VMEM((t_per_w, H), jnp.float32),
            pltpu.VMEM((t_per_w, H), jnp.float32),
            pltpu.SemaphoreType.DMA,
        ],
    )
    def k(osort_hbm, p0_hbm, p1_hbm, out_hbm, i0_v, i1_v, a_v, b_v, sem):
        wid = lax.axis_index("s") * mesh.num_cores + lax.axis_index("c")
        base = wid * t_per_w
        pltpu.sync_copy(p0_hbm.at[pl.ds(base, t_per_w)], i0_v)
        pltpu.sync_copy(p1_hbm.at[pl.ds(base, t_per_w)], i1_v)
        pltpu.async_copy(osort_hbm.at[i0_v], a_v, sem).wait()
        pltpu.async_copy(osort_hbm.at[i1_v], b_v, sem).wait()

        def row(r, carry):
            def col(c, carry2):
                sl = pl.ds(c * 16, 16)
                a_v[r, sl] = a_v[r, sl] + b_v[r, sl]
                return carry2
            return lax.fori_loop(0, csteps, col, carry)

        lax.fori_loop(0, t_per_w, row, 0)
        pltpu.sync_copy(a_v, out_hbm.at[pl.ds(base, t_per_w)])

    return k(osort, pos0, pos1)


def kernel(hidden_states, gate_w, gate_proj_w, up_proj_w, down_proj_w):
    batch, seq, hdim = hidden_states.shape
    x = hidden_states.reshape(-1, hdim)

    logits, ids8, ws8 = _run_router(x, gate_w)
    e2 = ids8[:, :TK]
    w2 = ws8[:, :TK]

    e_flat = e2.reshape(-1)
    onehot = (e_flat[:, None] == jnp.arange(NE, dtype=jnp.int32)[None, :])
    within = jnp.cumsum(onehot.astype(jnp.int32), axis=0)
    rank = jnp.take_along_axis(within, e_flat[:, None], axis=1)[:, 0] - 1
    counts = within[-1]
    padded = ((counts + BLK - 1) // BLK) * BLK
    pad_end = jnp.cumsum(padded)
    pad_off = pad_end - padded
    pos = pad_off[e_flat] + rank

    slot_token = jnp.arange(NSLOT, dtype=jnp.int32) // TK
    row_ids = jnp.zeros((PAD,), jnp.int32).at[pos].set(slot_token)
    w_pad = jnp.zeros((PAD,), jnp.float32).at[pos].set(w2.reshape(-1))
    block_expert = jnp.searchsorted(
        pad_end, jnp.arange(NB, dtype=jnp.int32) * BLK, side="right")
    block_expert = jnp.minimum(block_expert, NE - 1).astype(jnp.int32)

    xg = _sc_gather(x, row_ids)
    wcol = jnp.broadcast_to(w_pad[:, None], (PAD, 128))
    osort = _run_gmm(block_expert, xg, gate_proj_w, up_proj_w,
                     down_proj_w, wcol)
    pos2 = pos.reshape(S, TK)
    final = _sc_combine(osort, pos2[:, 0], pos2[:, 1])
    return final.reshape(batch, seq, hdim), logits


# speedup vs baseline: 1.1225x; 1.0449x over previous
import functools

import jax
import jax.numpy as jnp
from jax import lax
from jax.experimental import pallas as pl
from jax.experimental.pallas import tpu as pltpu
from jax.experimental.pallas import tpu_sc as plsc

NE = 8
TK = 2
BLK = 128
S = 2048
H = 768
I = 1536
NSLOT = S * TK
PAD = NSLOT + NE * BLK
NB = PAD // BLK


def _router_body(x_ref, gw_ref, logits_ref, ids_ref, ws_ref):
    x = x_ref[...]
    gw = gw_ref[...]
    logits = lax.dot_general(x, gw, (((1,), (1,)), ((), ())),
                             preferred_element_type=jnp.float32)
    logits_ref[...] = logits
    m = jnp.max(logits, axis=-1, keepdims=True)
    ex = jnp.exp(logits - m)
    p = ex / jnp.sum(ex, axis=-1, keepdims=True)
    idx = lax.broadcasted_iota(jnp.int32, p.shape, 1)
    m1 = jnp.max(p, axis=-1, keepdims=True)
    i1 = jnp.min(jnp.where(p == m1, idx, NE), axis=-1, keepdims=True)
    p2 = jnp.where(idx == i1, -1.0, p)
    m2 = jnp.max(p2, axis=-1, keepdims=True)
    i2 = jnp.min(jnp.where(p2 == m2, idx, NE), axis=-1, keepdims=True)
    denom = m1 + m2
    w1 = m1 / denom
    w2 = m2 / denom
    ids_ref[...] = jnp.where(idx == 0, i1, jnp.where(idx == 1, i2, 0))
    ws_ref[...] = jnp.where(idx == 0, w1, jnp.where(idx == 1, w2, 0.0))


def _run_router(x, gate_w):
    return pl.pallas_call(
        _router_body,
        out_shape=(
            jax.ShapeDtypeStruct((S, NE), jnp.float32),
            jax.ShapeDtypeStruct((S, NE), jnp.int32),
            jax.ShapeDtypeStruct((S, NE), jnp.float32),
        ),
    )(x, gate_w)


def _gmm_body(be_ref, xg_ref, gw_ref, uw_ref, dw_ref, wcol_ref, out_ref):
    del be_ref
    x = xg_ref[...]
    g = lax.dot_general(x, gw_ref[0], (((1,), (1,)), ((), ())),
                        preferred_element_type=jnp.float32)
    u = lax.dot_general(x, uw_ref[0], (((1,), (1,)), ((), ())),
                        preferred_element_type=jnp.float32)
    h = g * jax.nn.sigmoid(g) * u
    y = lax.dot_general(h, dw_ref[0], (((1,), (1,)), ((), ())),
                        preferred_element_type=jnp.float32)
    out_ref[...] = y * wcol_ref[...][:, :1]


def _run_gmm(block_expert, xg, gate_proj_w, up_proj_w, down_proj_w, wcol):
    grid_spec = pltpu.PrefetchScalarGridSpec(
        num_scalar_prefetch=1,
        grid=(NB,),
        in_specs=[
            pl.BlockSpec((BLK, H), lambda i, be: (i, 0)),
            pl.BlockSpec((1, I, H), lambda i, be: (be[i], 0, 0)),
            pl.BlockSpec((1, I, H), lambda i, be: (be[i], 0, 0)),
            pl.BlockSpec((1, H, I), lambda i, be: (be[i], 0, 0)),
            pl.BlockSpec((BLK, 128), lambda i, be: (i, 0)),
        ],
        out_specs=pl.BlockSpec((BLK, H), lambda i, be: (i, 0)),
    )
    return pl.pallas_call(
        _gmm_body,
        grid_spec=grid_spec,
        out_shape=jax.ShapeDtypeStruct((PAD, H), jnp.float32),
    )(block_expert, xg, gate_proj_w, up_proj_w, down_proj_w, wcol)


def _sc_gather(table, idx):
    mesh = plsc.VectorSubcoreMesh(core_axis_name="c", subcore_axis_name="s")
    nw = mesh.num_cores * mesh.num_subcores
    n = idx.shape[0]
    b_per_w = n // nw

    nch = 4
    ch = b_per_w // nch

    @functools.partial(
        pl.kernel,
        out_type=jax.ShapeDtypeStruct((n, H), jnp.float32),
        mesh=mesh,
        name="sc_gather_rows",
        scratch_types=[
            pltpu.VMEM((b_per_w,), jnp.int32),
            pltpu.VMEM((b_per_w, H), jnp.float32),
            pltpu.SemaphoreType.DMA,
            pltpu.SemaphoreType.DMA,
        ],
    )
    def k(table_hbm, idx_hbm, out_hbm, idx_v, rows_v, gsem, wsem):
        wid = lax.axis_index("s") * mesh.num_cores + lax.axis_index("c")
        base = wid * b_per_w
        pltpu.sync_copy(idx_hbm.at[pl.ds(base, b_per_w)], idx_v)
        gcp = [
            pltpu.async_copy(
                table_hbm.at[idx_v.at[pl.ds(c * ch, ch)]],
                rows_v.at[pl.ds(c * ch, ch)], gsem)
            for c in range(nch)
        ]
        wcp = []
        for c in range(nch):
            gcp[c].wait()
            wcp.append(pltpu.async_copy(
                rows_v.at[pl.ds(c * ch, ch)],
                out_hbm.at[pl.ds(base + c * ch, ch)], wsem))
        for c in range(nch):
            wcp[c].wait()

    return k(table, idx)


def _sc_combine(osort, pos0, pos1):
    mesh = plsc.VectorSubcoreMesh(core_axis_name="c", subcore_axis_name="s")
    nw = mesh.num_cores * mesh.num_subcores
    t_per_w = S // nw
    csteps = H // 16

    @functools.partial(
        pl.kernel,
        out_type=jax.ShapeDtypeStruct((S, H), jnp.float32),
        mesh=mesh,
        name="sc_combine_rows",
        scratch_types=[
            pltpu.VMEM((t_per_w,), jnp.int32),
            pltpu.VMEM((t_per_w,), jnp.int32),
            pltpu.VMEM((t_per_w, H), jnp.float32),
            pltpu.VMEM((t_per_w, H), jnp.float32),
            pltpu.SemaphoreType.DMA,
        ],
    )
    def k(osort_hbm, p0_hbm, p1_hbm, out_hbm, i0_v, i1_v, a_v, b_v, sem):
        wid = lax.axis_index("s") * mesh.num_cores + lax.axis_index("c")
        base = wid * t_per_w
        pltpu.sync_copy(p0_hbm.at[pl.ds(base, t_per_w)], i0_v)
        pltpu.sync_copy(p1_hbm.at[pl.ds(base, t_per_w)], i1_v)
        cp0 = pltpu.async_copy(osort_hbm.at[i0_v], a_v, sem)
        cp1 = pltpu.async_copy(osort_hbm.at[i1_v], b_v, sem)
        cp0.wait()
        cp1.wait()

        @plsc.parallel_loop(0, t_per_w * csteps, unroll=8)
        def _add(i):
            r = i // csteps
            c = i - r * csteps
            sl = pl.ds(c * 16, 16)
            a_v[r, sl] = a_v[r, sl] + b_v[r, sl]

        pltpu.sync_copy(a_v, out_hbm.at[pl.ds(base, t_per_w)])

    return k(osort, pos0, pos1)


def kernel(hidden_states, gate_w, gate_proj_w, up_proj_w, down_proj_w):
    batch, seq, hdim = hidden_states.shape
    x = hidden_states.reshape(-1, hdim)

    logits, ids8, ws8 = _run_router(x, gate_w)
    e2 = ids8[:, :TK]
    w2 = ws8[:, :TK]

    e_flat = e2.reshape(-1)
    onehot = (e_flat[:, None] == jnp.arange(NE, dtype=jnp.int32)[None, :])
    within = jnp.cumsum(onehot.astype(jnp.int32), axis=0)
    rank = jnp.take_along_axis(within, e_flat[:, None], axis=1)[:, 0] - 1
    counts = within[-1]
    padded = ((counts + BLK - 1) // BLK) * BLK
    pad_end = jnp.cumsum(padded)
    pad_off = pad_end - padded
    pos = pad_off[e_flat] + rank

    slot_token = jnp.arange(NSLOT, dtype=jnp.int32) // TK
    row_ids = jnp.zeros((PAD,), jnp.int32).at[pos].set(slot_token)
    w_pad = jnp.zeros((PAD,), jnp.float32).at[pos].set(w2.reshape(-1))
    block_expert = jnp.searchsorted(
        pad_end, jnp.arange(NB, dtype=jnp.int32) * BLK, side="right")
    block_expert = jnp.minimum(block_expert, NE - 1).astype(jnp.int32)

    xg = _sc_gather(x, row_ids)
    wcol = jnp.broadcast_to(w_pad[:, None], (PAD, 128))
    osort = _run_gmm(block_expert, xg, gate_proj_w, up_proj_w,
                     down_proj_w, wcol)
    pos2 = pos.reshape(S, TK)
    final = _sc_combine(osort, pos2[:, 0], pos2[:, 1])
    return final.reshape(batch, seq, hdim), logits
